# Initial kernel scaffold; baseline (speedup 1.0000x reference)
import jax
import jax.numpy as jnp
from jax.experimental import pallas as pl


def kernel(features, coords):
    raise NotImplementedError("write your pallas kernel here")


# speedup vs baseline: 1.6605x; 1.6605x over previous
import dataclasses
import functools

import jax
import jax.numpy as jnp
from jax.experimental import pallas as pl
from jax.experimental.pallas import tpu as pltpu
from jax.experimental.pallas import tpu_sc as plsc

B = 8
C = 64
N = 100000
R = 32
V = R * R * R

P = 4000
NCHUNK = N // P
LANES = 16


def _coords_kernel(coords_ref, nc_ref, idx_ref):
    c = coords_ref[0]
    mean = jnp.mean(c, axis=1, keepdims=True)
    cc = c - mean
    norm = jnp.sqrt(jnp.sum(cc * cc, axis=0, keepdims=True))
    maxn = jnp.max(norm)
    nc = cc / (2.0 * maxn) + 0.5
    ncr = jnp.clip(nc * float(R), 0.0, float(R - 1))
    nc_ref[...] = ncr.reshape(1, 3, N)
    vox = jnp.round(ncr).astype(jnp.int32)
    flat = (vox[0] * R + vox[1]) * R + vox[2]
    idx_ref[...] = flat.reshape(1, 1, N)


def _compute_norm_and_idx(coords):
    nc, idx3 = pl.pallas_call(
        _coords_kernel,
        grid=(B,),
        in_specs=[pl.BlockSpec((1, 3, N), lambda b: (b, 0, 0))],
        out_specs=[
            pl.BlockSpec((1, 3, N), lambda b: (b, 0, 0)),
            pl.BlockSpec((1, 1, N), lambda b: (b, 0, 0)),
        ],
        out_shape=[
            jax.ShapeDtypeStruct((B, 3, N), jnp.float32),
            jax.ShapeDtypeStruct((B, 1, N), jnp.int32),
        ],
    )(coords)
    return nc, idx3.reshape(B, N)


def _sc_scatter_kernel(feat_hbm, idx_hbm, out_hbm, acc, idxv, f0, f1):
    wid = jax.lax.axis_index("s") * 2 + jax.lax.axis_index("c")
    c0 = wid * 2
    c1 = c0 + 1
    ones = jnp.full((LANES,), 1.0, dtype=jnp.float32)

    @pl.loop(0, B)
    def _batch(b):
        @pl.loop(0, 3 * V, step=LANES)
        def _zero(i):
            acc.at[pl.ds(i, LANES)][...] = jnp.zeros((LANES,), jnp.float32)

        @pl.loop(0, NCHUNK)
        def _chunk(k):
            base = k * P
            pltpu.sync_copy(idx_hbm.at[pl.ds(b * N + base, P)], idxv)
            pltpu.sync_copy(
                feat_hbm.at[pl.ds((b * C + c0) * N + base, P)], f0
            )
            pltpu.sync_copy(
                feat_hbm.at[pl.ds((b * C + c1) * N + base, P)], f1
            )

            @pl.loop(0, P, step=LANES)
            def _group(j):
                iv = idxv.at[pl.ds(j, LANES)][...]
                x0 = f0.at[pl.ds(j, LANES)][...]
                x1 = f1.at[pl.ds(j, LANES)][...]
                plsc.addupdate_scatter(acc, [iv], x0)
                plsc.addupdate_scatter(acc, [iv + V], x1)
                plsc.addupdate_scatter(acc, [iv + 2 * V], ones)

        @pl.loop(0, V, step=LANES)
        def _div(j):
            cnt = jnp.maximum(acc.at[pl.ds(2 * V + j, LANES)][...], 1.0)
            acc.at[pl.ds(j, LANES)][...] = acc.at[pl.ds(j, LANES)][...] / cnt
            acc.at[pl.ds(V + j, LANES)][...] = (
                acc.at[pl.ds(V + j, LANES)][...] / cnt
            )

        pltpu.sync_copy(
            acc.at[pl.ds(0, V)], out_hbm.at[pl.ds((b * C + c0) * V, V)]
        )
        pltpu.sync_copy(
            acc.at[pl.ds(V, V)], out_hbm.at[pl.ds((b * C + c1) * V, V)]
        )


def _sc_scatter(features, idx):
    mesh = plsc.VectorSubcoreMesh(core_axis_name="c", subcore_axis_name="s")
    cp = pltpu.CompilerParams()
    if "needs_layout_passes" in pltpu.CompilerParams.__dataclass_fields__:
        cp = dataclasses.replace(cp, needs_layout_passes=False)
    fn = functools.partial(
        pl.kernel,
        compiler_params=cp,
        out_type=jax.ShapeDtypeStruct((B * C * V,), jnp.float32),
        mesh=mesh,
        scratch_types=[
            pltpu.VMEM((3 * V,), jnp.float32),
            pltpu.VMEM((P,), jnp.int32),
            pltpu.VMEM((P,), jnp.float32),
            pltpu.VMEM((P,), jnp.float32),
        ],
    )(_sc_scatter_kernel)
    return fn(features.reshape(B * C * N), idx.reshape(B * N))


def kernel(features, coords):
    norm_coords, idx = _compute_norm_and_idx(coords)
    sums = _sc_scatter(features, idx)
    return sums.reshape(B, C, R, R, R), norm_coords


# speedup vs baseline: 2.1279x; 1.2815x over previous
import dataclasses
import functools

import jax
import jax.numpy as jnp
from jax.experimental import pallas as pl
from jax.experimental.pallas import tpu as pltpu
from jax.experimental.pallas import tpu_sc as plsc

B = 8
C = 64
N = 100000
R = 32
V = R * R * R

P = 2000
NCHUNK = N // P
LANES = 16


def _coords_kernel(coords_ref, nc_ref, idx_ref):
    c = coords_ref[0]
    mean = jnp.mean(c, axis=1, keepdims=True)
    cc = c - mean
    norm = jnp.sqrt(jnp.sum(cc * cc, axis=0, keepdims=True))
    maxn = jnp.max(norm)
    nc = cc / (2.0 * maxn) + 0.5
    ncr = jnp.clip(nc * float(R), 0.0, float(R - 1))
    nc_ref[...] = ncr.reshape(1, 3, N)
    vox = jnp.round(ncr).astype(jnp.int32)
    flat = (vox[0] * R + vox[1]) * R + vox[2]
    idx_ref[...] = flat.reshape(1, 1, N)


def _compute_norm_and_idx(coords):
    nc, idx3 = pl.pallas_call(
        _coords_kernel,
        grid=(B,),
        in_specs=[pl.BlockSpec((1, 3, N), lambda b: (b, 0, 0))],
        out_specs=[
            pl.BlockSpec((1, 3, N), lambda b: (b, 0, 0)),
            pl.BlockSpec((1, 1, N), lambda b: (b, 0, 0)),
        ],
        out_shape=[
            jax.ShapeDtypeStruct((B, 3, N), jnp.float32),
            jax.ShapeDtypeStruct((B, 1, N), jnp.int32),
        ],
    )(coords)
    return nc, idx3.reshape(B, N)


def _sc_scatter_kernel(
    feat_hbm, idx_hbm, out_hbm, acc, idxv, f0, f1, idxv2, f02, f12, sem0, sem1
):
    wid = jax.lax.axis_index("s") * 2 + jax.lax.axis_index("c")
    c0 = wid * 2
    c1 = c0 + 1
    ones = jnp.full((LANES,), 1.0, dtype=jnp.float32)

    def start_chunk(b, k, ib, f0b, f1b, sem):
        base = k * P
        pltpu.async_copy(idx_hbm.at[pl.ds(b * N + base, P)], ib, sem)
        pltpu.async_copy(feat_hbm.at[pl.ds((b * C + c0) * N + base, P)], f0b, sem)
        pltpu.async_copy(feat_hbm.at[pl.ds((b * C + c1) * N + base, P)], f1b, sem)

    def wait_chunk(ib, f0b, f1b, sem):
        pltpu.make_async_copy(idx_hbm.at[pl.ds(0, P)], ib, sem).wait()
        pltpu.make_async_copy(idx_hbm.at[pl.ds(0, P)], f0b, sem).wait()
        pltpu.make_async_copy(idx_hbm.at[pl.ds(0, P)], f1b, sem).wait()

    def process(ib, f0b, f1b):
        @pl.loop(0, P, step=LANES)
        def _group(j):
            iv = ib.at[pl.ds(j, LANES)][...]
            x0 = f0b.at[pl.ds(j, LANES)][...]
            x1 = f1b.at[pl.ds(j, LANES)][...]
            plsc.addupdate_scatter(acc, [iv], x0)
            plsc.addupdate_scatter(acc, [iv + V], x1)
            plsc.addupdate_scatter(acc, [iv + 2 * V], ones)

    @pl.loop(0, B)
    def _batch(b):
        start_chunk(b, 0, idxv, f0, f1, sem0)

        @pl.loop(0, 3 * V, step=LANES)
        def _zero(i):
            acc.at[pl.ds(i, LANES)][...] = jnp.zeros((LANES,), jnp.float32)

        @pl.loop(0, NCHUNK, step=2)
        def _pair(k):
            start_chunk(b, k + 1, idxv2, f02, f12, sem1)
            wait_chunk(idxv, f0, f1, sem0)
            process(idxv, f0, f1)

            @pl.when(k + 2 < NCHUNK)
            def _prefetch():
                start_chunk(b, k + 2, idxv, f0, f1, sem0)

            wait_chunk(idxv2, f02, f12, sem1)
            process(idxv2, f02, f12)

        @pl.loop(0, V, step=LANES)
        def _div(j):
            cnt = jnp.maximum(acc.at[pl.ds(2 * V + j, LANES)][...], 1.0)
            acc.at[pl.ds(j, LANES)][...] = acc.at[pl.ds(j, LANES)][...] / cnt
            acc.at[pl.ds(V + j, LANES)][...] = (
                acc.at[pl.ds(V + j, LANES)][...] / cnt
            )

        pltpu.sync_copy(
            acc.at[pl.ds(0, V)], out_hbm.at[pl.ds((b * C + c0) * V, V)]
        )
        pltpu.sync_copy(
            acc.at[pl.ds(V, V)], out_hbm.at[pl.ds((b * C + c1) * V, V)]
        )


def _sc_scatter(features, idx):
    mesh = plsc.VectorSubcoreMesh(core_axis_name="c", subcore_axis_name="s")
    cp = pltpu.CompilerParams()
    if "needs_layout_passes" in pltpu.CompilerParams.__dataclass_fields__:
        cp = dataclasses.replace(cp, needs_layout_passes=False)
    fn = functools.partial(
        pl.kernel,
        compiler_params=cp,
        out_type=jax.ShapeDtypeStruct((B * C * V,), jnp.float32),
        mesh=mesh,
        scratch_types=[
            pltpu.VMEM((3 * V,), jnp.float32),
            pltpu.VMEM((P,), jnp.int32),
            pltpu.VMEM((P,), jnp.float32),
            pltpu.VMEM((P,), jnp.float32),
            pltpu.VMEM((P,), jnp.int32),
            pltpu.VMEM((P,), jnp.float32),
            pltpu.VMEM((P,), jnp.float32),
            pltpu.SemaphoreType.DMA,
            pltpu.SemaphoreType.DMA,
        ],
    )(_sc_scatter_kernel)
    return fn(features.reshape(B * C * N), idx.reshape(B * N))


def kernel(features, coords):
    norm_coords, idx = _compute_norm_and_idx(coords)
    sums = _sc_scatter(features, idx)
    return sums.reshape(B, C, R, R, R), norm_coords


# speedup vs baseline: 3.1181x; 1.4653x over previous
import dataclasses
import functools

import jax
import jax.numpy as jnp
from jax.experimental import pallas as pl
from jax.experimental.pallas import tpu as pltpu
from jax.experimental.pallas import tpu_sc as plsc

B = 8
C = 64
N = 100000
R = 32
V = R * R * R

P = 2000
NCHUNK = N // P
LANES = 16


def _coords_kernel(coords_ref, nc_ref, idx_ref):
    c = coords_ref[0]
    mean = jnp.mean(c, axis=1, keepdims=True)
    cc = c - mean
    norm = jnp.sqrt(jnp.sum(cc * cc, axis=0, keepdims=True))
    maxn = jnp.max(norm)
    nc = cc / (2.0 * maxn) + 0.5
    ncr = jnp.clip(nc * float(R), 0.0, float(R - 1))
    nc_ref[...] = ncr.reshape(1, 3, N)
    vox = jnp.round(ncr).astype(jnp.int32)
    flat = (vox[0] * R + vox[1]) * R + vox[2]
    idx_ref[...] = flat.reshape(1, 1, N)


def _compute_norm_and_idx(coords):
    nc, idx3 = pl.pallas_call(
        _coords_kernel,
        grid=(B,),
        in_specs=[pl.BlockSpec((1, 3, N), lambda b: (b, 0, 0))],
        out_specs=[
            pl.BlockSpec((1, 3, N), lambda b: (b, 0, 0)),
            pl.BlockSpec((1, 1, N), lambda b: (b, 0, 0)),
        ],
        out_shape=[
            jax.ShapeDtypeStruct((B, 3, N), jnp.float32),
            jax.ShapeDtypeStruct((B, 1, N), jnp.int32),
        ],
    )(coords)
    return nc, idx3.reshape(B, N)


def _sc_scatter_kernel(
    feat_hbm, idx_hbm, out_hbm, acc, idxv, f0, f1, idxv2, f02, f12, sem0, sem1
):
    wid = jax.lax.axis_index("s") * 2 + jax.lax.axis_index("c")
    c0 = wid * 2
    c1 = c0 + 1
    ones = jnp.full((LANES,), 1.0, dtype=jnp.float32)

    def start_chunk(b, k, ib, f0b, f1b, sem):
        base = k * P
        pltpu.async_copy(idx_hbm.at[pl.ds(b * N + base, P)], ib, sem)
        pltpu.async_copy(feat_hbm.at[pl.ds((b * C + c0) * N + base, P)], f0b, sem)
        pltpu.async_copy(feat_hbm.at[pl.ds((b * C + c1) * N + base, P)], f1b, sem)

    def wait_chunk(ib, f0b, f1b, sem):
        pltpu.make_async_copy(idx_hbm.at[pl.ds(0, P)], ib, sem).wait()
        pltpu.make_async_copy(idx_hbm.at[pl.ds(0, P)], f0b, sem).wait()
        pltpu.make_async_copy(idx_hbm.at[pl.ds(0, P)], f1b, sem).wait()

    def process(ib, f0b, f1b):
        @plsc.parallel_loop(0, P, step=LANES, unroll=4)
        def _group(j):
            iv = ib.at[pl.ds(j, LANES)][...]
            x0 = f0b.at[pl.ds(j, LANES)][...]
            x1 = f1b.at[pl.ds(j, LANES)][...]
            plsc.addupdate_scatter(acc, [iv], x0)
            plsc.addupdate_scatter(acc, [iv + V], x1)
            plsc.addupdate_scatter(acc, [iv + 2 * V], ones)

    @pl.loop(0, B)
    def _batch(b):
        start_chunk(b, 0, idxv, f0, f1, sem0)

        @plsc.parallel_loop(0, 3 * V, step=LANES, unroll=8)
        def _zero(i):
            acc.at[pl.ds(i, LANES)][...] = jnp.zeros((LANES,), jnp.float32)

        @pl.loop(0, NCHUNK, step=2)
        def _pair(k):
            start_chunk(b, k + 1, idxv2, f02, f12, sem1)
            wait_chunk(idxv, f0, f1, sem0)
            process(idxv, f0, f1)

            @pl.when(k + 2 < NCHUNK)
            def _prefetch():
                start_chunk(b, k + 2, idxv, f0, f1, sem0)

            wait_chunk(idxv2, f02, f12, sem1)
            process(idxv2, f02, f12)

        @plsc.parallel_loop(0, V, step=LANES, unroll=4)
        def _div(j):
            cnt = jnp.maximum(acc.at[pl.ds(2 * V + j, LANES)][...], 1.0)
            acc.at[pl.ds(j, LANES)][...] = acc.at[pl.ds(j, LANES)][...] / cnt
            acc.at[pl.ds(V + j, LANES)][...] = (
                acc.at[pl.ds(V + j, LANES)][...] / cnt
            )

        pltpu.sync_copy(
            acc.at[pl.ds(0, V)], out_hbm.at[pl.ds((b * C + c0) * V, V)]
        )
        pltpu.sync_copy(
            acc.at[pl.ds(V, V)], out_hbm.at[pl.ds((b * C + c1) * V, V)]
        )


def _sc_scatter(features, idx):
    mesh = plsc.VectorSubcoreMesh(core_axis_name="c", subcore_axis_name="s")
    cp = pltpu.CompilerParams()
    if "needs_layout_passes" in pltpu.CompilerParams.__dataclass_fields__:
        cp = dataclasses.replace(cp, needs_layout_passes=False)
    fn = functools.partial(
        pl.kernel,
        compiler_params=cp,
        out_type=jax.ShapeDtypeStruct((B * C * V,), jnp.float32),
        mesh=mesh,
        scratch_types=[
            pltpu.VMEM((3 * V,), jnp.float32),
            pltpu.VMEM((P,), jnp.int32),
            pltpu.VMEM((P,), jnp.float32),
            pltpu.VMEM((P,), jnp.float32),
            pltpu.VMEM((P,), jnp.int32),
            pltpu.VMEM((P,), jnp.float32),
            pltpu.VMEM((P,), jnp.float32),
            pltpu.SemaphoreType.DMA,
            pltpu.SemaphoreType.DMA,
        ],
    )(_sc_scatter_kernel)
    return fn(features.reshape(B * C * N), idx.reshape(B * N))


def kernel(features, coords):
    norm_coords, idx = _compute_norm_and_idx(coords)
    sums = _sc_scatter(features, idx)
    return sums.reshape(B, C, R, R, R), norm_coords
